# Initial kernel scaffold; baseline (speedup 1.0000x reference)
import jax
import jax.numpy as jnp
from jax.experimental import pallas as pl


def kernel(x, edge_index, W1a, b1a, W1b, b1b, W2a, b2a, W2b, b2b, Wfc, bfc):
    raise NotImplementedError("write your pallas kernel here")


# speedup vs baseline: 2.8200x; 2.8200x over previous
import functools
import jax
import jax.numpy as jnp
from jax import lax
from jax.experimental import pallas as pl
from jax.experimental.pallas import tpu as pltpu, tpu_sc as plsc

_N = 10000
_E = 320000
_D = 128
_K = 128
_NC = 2
_NS = 16
_NP = 10240
_EP = _NC * _NS * _NP
_CPT = _NP // _K
_RPT = _NP // _NS


def _seg_sum_body(h_hbm, src_hbm, dst_hbm, zeros_hbm, out_hbm,
                  shared, src_v, dst_v, rows_v, sem):
    cid = lax.axis_index("c")
    sid = lax.axis_index("s")
    wid = cid * _NS + sid

    pltpu.sync_copy(zeros_hbm, shared.at[pl.ds(sid * _RPT, _RPT)])

    base = wid * _CPT
    pltpu.sync_copy(src_hbm.at[pl.ds(base, _CPT)], src_v)
    pltpu.sync_copy(dst_hbm.at[pl.ds(base, _CPT)], dst_v)
    plsc.subcore_barrier()

    def body(j, carry):
        pltpu.async_copy(h_hbm.at[src_v.at[j]], rows_v, sem).wait()
        pltpu.sync_copy(rows_v, shared.at[dst_v.at[j]], add=True)
        return carry

    lax.fori_loop(0, _CPT, body, 0)
    plsc.subcore_barrier()

    pltpu.sync_copy(shared.at[pl.ds(sid * _RPT, _RPT)],
                    out_hbm.at[pl.ds(cid * _NP + sid * _RPT, _RPT)])


_seg_sum = pl.kernel(
    _seg_sum_body,
    mesh=plsc.VectorSubcoreMesh(core_axis_name="c", subcore_axis_name="s"),
    out_type=jax.ShapeDtypeStruct((_NC * _NP, _D), jnp.float32),
    scratch_types=[
        pltpu.VMEM_SHARED((_NP, _D), jnp.float32),
        pltpu.VMEM((_CPT, _K), jnp.int32),
        pltpu.VMEM((_CPT, _K), jnp.int32),
        pltpu.VMEM((_K, _D), jnp.float32),
        pltpu.SemaphoreType.DMA,
    ],
)


_R = 1000


def _mlp_body(h_ref, agg_ref, wa_ref, ba_ref, wb_ref, bb_ref, o_ref):
    z = h_ref[...] + agg_ref[0] + agg_ref[1]
    z = jnp.maximum(jnp.dot(z, wa_ref[...],
                            preferred_element_type=jnp.float32) + ba_ref[...], 0.0)
    z = jnp.maximum(jnp.dot(z, wb_ref[...],
                            preferred_element_type=jnp.float32) + bb_ref[...], 0.0)
    o_ref[...] = z


_mlp = pl.pallas_call(
    _mlp_body,
    grid=(_N // _R,),
    in_specs=[
        pl.BlockSpec((_R, _D), lambda i: (i, 0)),
        pl.BlockSpec((_NC, _R, _D), lambda i: (0, i, 0)),
        pl.BlockSpec((_D, _D), lambda i: (0, 0)),
        pl.BlockSpec((1, _D), lambda i: (0, 0)),
        pl.BlockSpec((_D, _D), lambda i: (0, 0)),
        pl.BlockSpec((1, _D), lambda i: (0, 0)),
    ],
    out_specs=pl.BlockSpec((_R, _D), lambda i: (i, 0)),
    out_shape=jax.ShapeDtypeStruct((_N, _D), jnp.float32),
)


def _mlp_fc_body(h_ref, agg_ref, wa_ref, ba_ref, wb_ref, bb_ref,
                 wfc_ref, bfc_ref, o_ref):
    z = h_ref[...] + agg_ref[0] + agg_ref[1]
    z = jnp.maximum(jnp.dot(z, wa_ref[...],
                            preferred_element_type=jnp.float32) + ba_ref[...], 0.0)
    z = jnp.maximum(jnp.dot(z, wb_ref[...],
                            preferred_element_type=jnp.float32) + bb_ref[...], 0.0)
    s = jnp.sum(z * wfc_ref[...], axis=1, keepdims=True) + bfc_ref[0, 0]
    o_ref[...] = jax.nn.sigmoid(s)


_mlp_fc = pl.pallas_call(
    _mlp_fc_body,
    grid=(_N // _R,),
    in_specs=[
        pl.BlockSpec((_R, _D), lambda i: (i, 0)),
        pl.BlockSpec((_NC, _R, _D), lambda i: (0, i, 0)),
        pl.BlockSpec((_D, _D), lambda i: (0, 0)),
        pl.BlockSpec((1, _D), lambda i: (0, 0)),
        pl.BlockSpec((_D, _D), lambda i: (0, 0)),
        pl.BlockSpec((1, _D), lambda i: (0, 0)),
        pl.BlockSpec((1, _D), lambda i: (0, 0)),
        pl.BlockSpec((1, 1), lambda i: (0, 0), memory_space=pltpu.SMEM),
    ],
    out_specs=pl.BlockSpec((_R, 1), lambda i: (i, 0)),
    out_shape=jax.ShapeDtypeStruct((_N, 1), jnp.float32),
)


def _pad_edges(edge_index):
    pad = _EP - _E
    src = jnp.concatenate(
        [edge_index[0], jnp.zeros((pad,), jnp.int32)]).reshape(_EP // _K, _K)
    dst = jnp.concatenate(
        [edge_index[1], jnp.full((pad,), _N, jnp.int32)]).reshape(_EP // _K, _K)
    return src, dst


@jax.jit
def kernel(x, edge_index, W1a, b1a, W1b, b1b, W2a, b2a, W2b, b2b, Wfc, bfc):
    src, dst = _pad_edges(edge_index)
    zeros = jnp.zeros((_RPT, _D), jnp.float32)

    agg1 = _seg_sum(x, src, dst, zeros).reshape(_NC, _NP, _D)[:, :_N]
    h1 = _mlp(x, agg1, W1a, b1a.reshape(1, _D), W1b, b1b.reshape(1, _D))
    agg2 = _seg_sum(h1, src, dst, zeros).reshape(_NC, _NP, _D)[:, :_N]
    out = _mlp_fc(h1, agg2, W2a, b2a.reshape(1, _D), W2b, b2b.reshape(1, _D),
                  Wfc.reshape(1, _D), bfc.reshape(1, 1))
    return out
